# baseline (speedup 1.0000x reference)
import functools

import jax
import jax.numpy as jnp
from jax import lax
from jax.experimental import pallas as pl
from jax.experimental.pallas import tpu as pltpu
from jax.experimental.pallas import tpu_sc as plsc

N_GRID = 100000
N_MESH = 5882
D = 128

NC, NS = 2, 16
NW = NC * NS
GRP = 128
GPW = 13
E_HALF = NW * GPW * GRP
E_PAD = 2 * E_HALF
RA = 2048
GA = E_HALF // RA
S_PAD = 5888
ROWS_PER_TILE = S_PAD // NS
XB = (N_GRID - 1) // RA


def _dot(a, b):
    return jnp.dot(a.astype(jnp.bfloat16), b.astype(jnp.bfloat16),
                   preferred_element_type=jnp.float32)


def _ln(y, g, b):
    mu = jnp.mean(y, axis=-1, keepdims=True)
    var = jnp.mean(y * y, axis=-1, keepdims=True) - mu * mu
    return (y - mu) * jax.lax.rsqrt(var + 1e-5) * g + b


def _stage_a_body(x_ref, ea_ref,
                  eW1, eb1, eW2, eb2, eg, ebt,
                  fW1f, fW1a, fb1, fWdx, fWda, fWdh, cd, fgs, cbe,
                  nW1a, nW1b, nb1, nW2a, nW2b, nb2a, nb2b, nga, ngb,
                  nba, nbb,
                  ma_ref, mb_ref):
    f32 = jnp.float32
    x = x_ref[...]
    ea = ea_ref[...]
    h = jnp.maximum(_dot(x, eW1[...]) + eb1[...], 0.0)
    x_hat = _ln(x + _dot(h, eW2[...]) + eb2[...], eg[...], ebt[...])
    hf = jnp.maximum(_dot(x_hat, fW1f[...]) + _dot(ea, fW1a[...])
                     + fb1[...], 0.0)
    dcol = (_dot(x_hat, fWdx[...]) + _dot(ea, fWda[...])
            + _dot(hf, fWdh[...]) + cd[...])
    t = dcol * jax.lax.rsqrt(dcol * dcol + 1e-5)
    e_out = t * fgs[...] + (cbe[...] + ea)
    h2 = jnp.maximum(_dot(x_hat, nW1a[...]) + _dot(e_out, nW1b[...])
                     + nb1[...], 0.0)
    y2a = x_hat + _dot(h2, nW2a[...]) + nb2a[...]
    y2b = e_out + _dot(h2, nW2b[...]) + nb2b[...]
    mu = (jnp.sum(y2a, axis=-1, keepdims=True)
          + jnp.sum(y2b, axis=-1, keepdims=True)) * (1.0 / 130.0)
    var = (jnp.sum(y2a * y2a, axis=-1, keepdims=True)
           + jnp.sum(y2b * y2b, axis=-1, keepdims=True)) * (1.0 / 130.0) - mu * mu
    inv = jax.lax.rsqrt(var + 1e-5)
    ma_ref[...] = (y2a - mu) * inv * nga[...] + nba[...]
    mbv = (y2b - mu) * inv * ngb[...] + nbb[...]
    rows = mbv.shape[0]
    mb_ref[...] = jnp.concatenate(
        [mbv, jnp.ones((rows, 1), f32), jnp.zeros((rows, 13), f32)], axis=1)


def _stage_c_body(sa0_ref, sb0_ref, sa1_ref, sb1_ref,
                  W12a, W12b, b12, rW2a, rW2b, rb2, W22, b22, g2, bt2,
                  out_ref):
    sa = (sa0_ref[0:S_PAD, :] + sa0_ref[S_PAD:2 * S_PAD, :]
          + sa1_ref[0:S_PAD, :] + sa1_ref[S_PAD:2 * S_PAD, :])
    sb = (sb0_ref[0:S_PAD, :] + sb0_ref[S_PAD:2 * S_PAD, :]
          + sb1_ref[0:S_PAD, :] + sb1_ref[S_PAD:2 * S_PAD, :])
    cnt = sb[:, 2:3]
    denom = 1.0 / jnp.maximum(cnt, 1.0)
    agg_a = sa * denom
    agg_b = sb[:, 0:2] * denom
    h3 = jnp.maximum(_dot(agg_a, W12a[...]) + _dot(agg_b, W12b[...])
                     + b12[...], 0.0)
    y3 = (_dot(agg_a, rW2a[...]) + _dot(agg_b, rW2b[...]) + rb2[...]
          + _dot(h3, W22[...]) + b22[...])
    out = _ln(y3, g2[...], bt2[...])
    out_ref[...] = out[0:N_MESH, :]


def _scatter_body(ma_hbm, mbf_hbm, idx_hbm, za_hbm,
                  sa_out, sb_out,
                  idx_v, bn_c, a_v, b_v, sa_sh, sb_sh):
    cid = lax.axis_index("c")
    sid = lax.axis_index("s")
    wid = cid * NS + sid

    arow = sid * ROWS_PER_TILE
    pltpu.sync_copy(za_hbm.at[pl.ds(arow, ROWS_PER_TILE)],
                    sa_sh.at[pl.ds(arow, ROWS_PER_TILE)])
    pltpu.sync_copy(za_hbm.at[pl.ds(arow, ROWS_PER_TILE)],
                    sb_sh.at[pl.ds(arow, ROWS_PER_TILE)])
    plsc.subcore_barrier()

    ebase = wid * GPW * GRP
    pltpu.sync_copy(idx_hbm.at[wid], idx_v)
    for c in range(GPW):
        off = ebase + c * GRP
        pltpu.sync_copy(ma_hbm.at[pl.ds(off, GRP)], a_v)
        pltpu.sync_copy(mbf_hbm.at[pl.ds(off * 16, GRP * 16)], bn_c)
        for r in range(GRP):
            b_v.at[r][pl.ds(0, 16)] = bn_c[pl.ds(16 * r, 16)]
        pltpu.sync_copy(a_v, sa_sh.at[idx_v.at[c]], add=True)
        pltpu.sync_copy(b_v, sb_sh.at[idx_v.at[c]], add=True)
    plsc.subcore_barrier()

    obase = cid * S_PAD + sid * ROWS_PER_TILE
    pltpu.sync_copy(sa_sh.at[pl.ds(arow, ROWS_PER_TILE)],
                    sa_out.at[pl.ds(obase, ROWS_PER_TILE)])
    pltpu.sync_copy(sb_sh.at[pl.ds(arow, ROWS_PER_TILE)],
                    sb_out.at[pl.ds(obase, ROWS_PER_TILE)])


def _row(v):
    return v.reshape(1, -1)


def _full(shape):
    return pl.BlockSpec(shape, lambda *_: (0,) * len(shape))


def _stage_a_call(x, ea, wts, half, interpret=False):
    base = half * GA
    rowspec = pl.BlockSpec(
        (RA, D), lambda i: (jnp.minimum(base + i, XB), 0))
    easpec = pl.BlockSpec(
        (RA, 2), lambda i: (jnp.minimum(base + i, XB), 0))
    wspecs = [_full(w.shape) for w in wts]
    return pl.pallas_call(
        _stage_a_body,
        grid=(GA,),
        in_specs=[rowspec, easpec] + wspecs,
        out_specs=[pl.BlockSpec((RA, D), lambda i: (i, 0)),
                   pl.BlockSpec((RA, 16), lambda i: (i, 0))],
        out_shape=[jax.ShapeDtypeStruct((E_HALF, D), jnp.float32),
                   jax.ShapeDtypeStruct((E_HALF, 16), jnp.float32)],
        interpret=interpret,
    )(x, ea, *wts)


def _stage_c_call(parts, wts, interpret=False):
    wspecs = [_full(w.shape) for w in wts]
    return pl.pallas_call(
        _stage_c_body,
        in_specs=[_full((2 * S_PAD, D))] * 4 + wspecs,
        out_specs=pl.BlockSpec((N_MESH, D), lambda: (0, 0)),
        out_shape=jax.ShapeDtypeStruct((N_MESH, D), jnp.float32),
        interpret=interpret,
    )(*parts, *wts)


@functools.cache
def _scatter_call():
    return pl.kernel(
        _scatter_body,
        mesh=plsc.VectorSubcoreMesh(core_axis_name="c", subcore_axis_name="s"),
        out_type=[jax.ShapeDtypeStruct((NC * S_PAD, D), jnp.float32),
                  jax.ShapeDtypeStruct((NC * S_PAD, D), jnp.float32)],
        scratch_types=[pltpu.VMEM((GPW, GRP), jnp.int32),
                       pltpu.VMEM((GRP * 16,), jnp.float32),
                       pltpu.VMEM((GRP, D), jnp.float32),
                       pltpu.VMEM((GRP, D), jnp.float32),
                       pltpu.VMEM_SHARED((S_PAD, D), jnp.float32),
                       pltpu.VMEM_SHARED((S_PAD, D), jnp.float32)],
    )


def _make_a_weights(enc, edge_p, nm1):
    half = jnp.float32(0.5)
    frW = edge_p['res_W']
    fW2 = edge_p['W2']
    fWdx = ((frW[0:D, 0] - frW[0:D, 1]) * half).reshape(D, 1)
    fWda = ((frW[2 * D:2 * D + 2, 0] - frW[2 * D:2 * D + 2, 1]) * half).reshape(2, 1)
    fWdh = ((fW2[:, 0] - fW2[:, 1]) * half).reshape(D, 1)
    cy = edge_p['res_b'] + edge_p['b2']
    cd = ((cy[0] - cy[1]) * half).reshape(1, 1)
    fgs = (edge_p['ln_g'] * jnp.array([1.0, -1.0], jnp.float32)).reshape(1, 2)
    cbe = _row(edge_p['ln_b'])
    return [
        enc['W1'], _row(enc['b1']), enc['W2'], _row(enc['b2']),
        _row(enc['ln_g']), _row(enc['ln_b']),
        edge_p['W1'][0:D], edge_p['W1'][2 * D:2 * D + 2], _row(edge_p['b1']),
        fWdx, fWda, fWdh, cd, fgs, cbe,
        nm1['W1'][0:D], nm1['W1'][D:D + 2], _row(nm1['b1']),
        nm1['W2'][:, 0:D], nm1['W2'][:, D:D + 2],
        _row(nm1['b2'][0:D]), _row(nm1['b2'][D:D + 2]),
        _row(nm1['ln_g'][0:D]), _row(nm1['ln_g'][D:D + 2]),
        _row(nm1['ln_b'][0:D]), _row(nm1['ln_b'][D:D + 2]),
    ]


def _make_c_weights(nm2):
    return [
        nm2['W1'][D:2 * D], nm2['W1'][2 * D:2 * D + 2], _row(nm2['b1']),
        nm2['res_W'][D:2 * D], nm2['res_W'][2 * D:2 * D + 2],
        _row(nm2['res_b']), nm2['W2'], _row(nm2['b2']),
        _row(nm2['ln_g']), _row(nm2['ln_b']),
    ]


def kernel(x, edge_index, edge_attr, enc, edge_p, nm1, nm2):
    a_wts = _make_a_weights(enc, edge_p, nm1)
    c_wts = _make_c_weights(nm2)

    col = edge_index[1].astype(jnp.int32) - N_GRID
    idx = jnp.concatenate(
        [col, jnp.full((E_PAD - N_GRID,), N_MESH, jnp.int32)]).reshape(
            2, NW, GPW, GRP)
    za = jnp.zeros((S_PAD, D), jnp.float32)

    parts = []
    for h in range(2):
        ma, mb = _stage_a_call(x, edge_attr, a_wts, h)
        sa, sb = _scatter_call()(ma, mb.reshape(-1), idx[h], za)
        parts += [sa, sb]
    return _stage_c_call(parts, c_wts)


# speedup vs baseline: 1.0994x; 1.0994x over previous
import functools

import jax
import jax.numpy as jnp
from jax import lax
from jax.experimental import pallas as pl
from jax.experimental.pallas import tpu as pltpu
from jax.experimental.pallas import tpu_sc as plsc

N_GRID = 100000
N_MESH = 5882
D = 128

NC, NS = 2, 16
NW = NC * NS
GRP = 128
GPW = 13
E_HALF = NW * GPW * GRP
E_PAD = 2 * E_HALF
RA = 4096
GA = E_HALF // RA
S_PAD = 5888
ROWS_PER_TILE = S_PAD // NS
XB = (N_GRID - 1) // RA


def _dot(a, b):
    return jnp.dot(a.astype(jnp.bfloat16), b.astype(jnp.bfloat16),
                   preferred_element_type=jnp.float32)


def _ln(y, g, b):
    mu = jnp.mean(y, axis=-1, keepdims=True)
    var = jnp.mean(y * y, axis=-1, keepdims=True) - mu * mu
    return (y - mu) * jax.lax.rsqrt(var + 1e-5) * g + b


def _stage_a_body(x_ref, ea_ref,
                  eW1, eb1, eW2, eb2, eg, ebt,
                  fW1f, fW1a, fb1, fWdx, fWda, fWdh, cd, fgs, cbe,
                  nW1a, nW1b, nb1, nW2a, nW2b, nb2a, nb2b, nga, ngb,
                  nba, nbb,
                  ma_ref, mb_ref):
    f32 = jnp.float32
    x = x_ref[...]
    ea = ea_ref[...]
    h = jnp.maximum(_dot(x, eW1[...]) + eb1[...], 0.0)
    x_hat = _ln(x + _dot(h, eW2[...]) + eb2[...], eg[...], ebt[...])
    hf = jnp.maximum(_dot(x_hat, fW1f[...]) + _dot(ea, fW1a[...])
                     + fb1[...], 0.0)
    dcol = (_dot(x_hat, fWdx[...]) + _dot(ea, fWda[...])
            + _dot(hf, fWdh[...]) + cd[...])
    t = dcol * jax.lax.rsqrt(dcol * dcol + 1e-5)
    e_out = t * fgs[...] + (cbe[...] + ea)
    h2 = jnp.maximum(_dot(x_hat, nW1a[...]) + _dot(e_out, nW1b[...])
                     + nb1[...], 0.0)
    y2a = x_hat + _dot(h2, nW2a[...]) + nb2a[...]
    y2b = e_out + _dot(h2, nW2b[...]) + nb2b[...]
    mu = (jnp.sum(y2a, axis=-1, keepdims=True)
          + jnp.sum(y2b, axis=-1, keepdims=True)) * (1.0 / 130.0)
    var = (jnp.sum(y2a * y2a, axis=-1, keepdims=True)
           + jnp.sum(y2b * y2b, axis=-1, keepdims=True)) * (1.0 / 130.0) - mu * mu
    inv = jax.lax.rsqrt(var + 1e-5)
    ma_ref[...] = (y2a - mu) * inv * nga[...] + nba[...]
    mbv = (y2b - mu) * inv * ngb[...] + nbb[...]
    rows = mbv.shape[0]
    mb_ref[:, 0:8] = jnp.concatenate(
        [mbv, jnp.ones((rows, 1), f32), jnp.zeros((rows, 5), f32)], axis=1)


def _stage_c_body(sa0_ref, sb0_ref, sa1_ref, sb1_ref,
                  W12a, W12b, b12, rW2a, rW2b, rb2, W22, b22, g2, bt2,
                  out_ref):
    sa = (sa0_ref[0:S_PAD, :] + sa0_ref[S_PAD:2 * S_PAD, :]
          + sa1_ref[0:S_PAD, :] + sa1_ref[S_PAD:2 * S_PAD, :])
    sb = (sb0_ref[0:S_PAD, :] + sb0_ref[S_PAD:2 * S_PAD, :]
          + sb1_ref[0:S_PAD, :] + sb1_ref[S_PAD:2 * S_PAD, :])
    cnt = sb[:, 2:3]
    denom = 1.0 / jnp.maximum(cnt, 1.0)
    agg_a = sa * denom
    agg_b = sb[:, 0:2] * denom
    h3 = jnp.maximum(_dot(agg_a, W12a[...]) + _dot(agg_b, W12b[...])
                     + b12[...], 0.0)
    y3 = (_dot(agg_a, rW2a[...]) + _dot(agg_b, rW2b[...]) + rb2[...]
          + _dot(h3, W22[...]) + b22[...])
    out = _ln(y3, g2[...], bt2[...])
    out_ref[...] = out[0:N_MESH, :]


def _scatter_body(ma_hbm, mb_hbm, idx_hbm, za_hbm,
                  sa_out, sb_out,
                  idx_v, a_v, b_v, sa_sh, sb_sh):
    cid = lax.axis_index("c")
    sid = lax.axis_index("s")
    wid = cid * NS + sid

    arow = sid * ROWS_PER_TILE
    pltpu.sync_copy(za_hbm.at[pl.ds(arow, ROWS_PER_TILE)],
                    sa_sh.at[pl.ds(arow, ROWS_PER_TILE)])
    pltpu.sync_copy(za_hbm.at[pl.ds(arow, ROWS_PER_TILE)],
                    sb_sh.at[pl.ds(arow, ROWS_PER_TILE)])
    plsc.subcore_barrier()

    ebase = wid * GPW * GRP
    pltpu.sync_copy(idx_hbm.at[wid], idx_v)
    for c in range(GPW):
        off = ebase + c * GRP
        pltpu.sync_copy(ma_hbm.at[pl.ds(off, GRP)], a_v)
        pltpu.sync_copy(mb_hbm.at[pl.ds(off, GRP)], b_v)
        pltpu.sync_copy(a_v, sa_sh.at[idx_v.at[c]], add=True)
        pltpu.sync_copy(b_v, sb_sh.at[idx_v.at[c]], add=True)
    plsc.subcore_barrier()

    obase = cid * S_PAD + sid * ROWS_PER_TILE
    pltpu.sync_copy(sa_sh.at[pl.ds(arow, ROWS_PER_TILE)],
                    sa_out.at[pl.ds(obase, ROWS_PER_TILE)])
    pltpu.sync_copy(sb_sh.at[pl.ds(arow, ROWS_PER_TILE)],
                    sb_out.at[pl.ds(obase, ROWS_PER_TILE)])


def _row(v):
    return v.reshape(1, -1)


def _full(shape):
    return pl.BlockSpec(shape, lambda *_: (0,) * len(shape))


def _stage_a_call(x, ea, wts, half, interpret=False):
    base = half * GA
    rowspec = pl.BlockSpec(
        (RA, D), lambda i: (jnp.minimum(base + i, XB), 0))
    easpec = pl.BlockSpec(
        (RA, 2), lambda i: (jnp.minimum(base + i, XB), 0))
    wspecs = [_full(w.shape) for w in wts]
    return pl.pallas_call(
        _stage_a_body,
        grid=(GA,),
        in_specs=[rowspec, easpec] + wspecs,
        out_specs=[pl.BlockSpec((RA, D), lambda i: (i, 0)),
                   pl.BlockSpec((RA, D), lambda i: (i, 0))],
        out_shape=[jax.ShapeDtypeStruct((E_HALF, D), jnp.float32),
                   jax.ShapeDtypeStruct((E_HALF, D), jnp.float32)],
        interpret=interpret,
    )(x, ea, *wts)


def _stage_c_call(parts, wts, interpret=False):
    wspecs = [_full(w.shape) for w in wts]
    return pl.pallas_call(
        _stage_c_body,
        in_specs=[_full((2 * S_PAD, D))] * 4 + wspecs,
        out_specs=pl.BlockSpec((N_MESH, D), lambda: (0, 0)),
        out_shape=jax.ShapeDtypeStruct((N_MESH, D), jnp.float32),
        interpret=interpret,
    )(*parts, *wts)


@functools.cache
def _scatter_call():
    return pl.kernel(
        _scatter_body,
        mesh=plsc.VectorSubcoreMesh(core_axis_name="c", subcore_axis_name="s"),
        out_type=[jax.ShapeDtypeStruct((NC * S_PAD, D), jnp.float32),
                  jax.ShapeDtypeStruct((NC * S_PAD, D), jnp.float32)],
        scratch_types=[pltpu.VMEM((GPW, GRP), jnp.int32),
                       pltpu.VMEM((GRP, D), jnp.float32),
                       pltpu.VMEM((GRP, D), jnp.float32),
                       pltpu.VMEM_SHARED((S_PAD, D), jnp.float32),
                       pltpu.VMEM_SHARED((S_PAD, D), jnp.float32)],
    )


def _make_a_weights(enc, edge_p, nm1):
    half = jnp.float32(0.5)
    frW = edge_p['res_W']
    fW2 = edge_p['W2']
    fWdx = ((frW[0:D, 0] - frW[0:D, 1]) * half).reshape(D, 1)
    fWda = ((frW[2 * D:2 * D + 2, 0] - frW[2 * D:2 * D + 2, 1]) * half).reshape(2, 1)
    fWdh = ((fW2[:, 0] - fW2[:, 1]) * half).reshape(D, 1)
    cy = edge_p['res_b'] + edge_p['b2']
    cd = ((cy[0] - cy[1]) * half).reshape(1, 1)
    fgs = (edge_p['ln_g'] * jnp.array([1.0, -1.0], jnp.float32)).reshape(1, 2)
    cbe = _row(edge_p['ln_b'])
    return [
        enc['W1'], _row(enc['b1']), enc['W2'], _row(enc['b2']),
        _row(enc['ln_g']), _row(enc['ln_b']),
        edge_p['W1'][0:D], edge_p['W1'][2 * D:2 * D + 2], _row(edge_p['b1']),
        fWdx, fWda, fWdh, cd, fgs, cbe,
        nm1['W1'][0:D], nm1['W1'][D:D + 2], _row(nm1['b1']),
        nm1['W2'][:, 0:D], nm1['W2'][:, D:D + 2],
        _row(nm1['b2'][0:D]), _row(nm1['b2'][D:D + 2]),
        _row(nm1['ln_g'][0:D]), _row(nm1['ln_g'][D:D + 2]),
        _row(nm1['ln_b'][0:D]), _row(nm1['ln_b'][D:D + 2]),
    ]


def _make_c_weights(nm2):
    return [
        nm2['W1'][D:2 * D], nm2['W1'][2 * D:2 * D + 2], _row(nm2['b1']),
        nm2['res_W'][D:2 * D], nm2['res_W'][2 * D:2 * D + 2],
        _row(nm2['res_b']), nm2['W2'], _row(nm2['b2']),
        _row(nm2['ln_g']), _row(nm2['ln_b']),
    ]


def kernel(x, edge_index, edge_attr, enc, edge_p, nm1, nm2):
    a_wts = _make_a_weights(enc, edge_p, nm1)
    c_wts = _make_c_weights(nm2)

    col = edge_index[1].astype(jnp.int32) - N_GRID
    idx = jnp.concatenate(
        [col, jnp.full((E_PAD - N_GRID,), N_MESH, jnp.int32)]).reshape(
            2, NW, GPW, GRP)
    za = jnp.zeros((S_PAD, D), jnp.float32)

    parts = []
    for h in range(2):
        ma, mb = _stage_a_call(x, edge_attr, a_wts, h)
        sa, sb = _scatter_call()(ma, mb, idx[h], za)
        parts += [sa, sb]
    return _stage_c_call(parts, c_wts)
